# baseline (speedup 1.0000x reference)
import functools

import jax
import jax.numpy as jnp
from jax import lax
from jax.experimental import pallas as pl
from jax.experimental.pallas import tpu as pltpu
from jax.experimental.pallas import tpu_sc as plsc

_NC = 2
_NS = 16
_L = 16
_NW = _NC * _NS

_CP = 4
_CH = 8192
_VP = 100096
_VS = _VP // _NW

_sc_params = pltpu.CompilerParams(
    needs_layout_passes=False, use_tc_tiling_on_sc=False)
_mesh = plsc.VectorSubcoreMesh(
    core_axis_name="c", subcore_axis_name="s",
    num_cores=_NC, num_subcores=_NS)


def _proj_body(embT_ref, w_ref, out_ref):
    out_ref[...] = lax.dot_general(
        w_ref[...], embT_ref[...], (((0,), (0,)), ((), ())),
        preferred_element_type=jnp.float32)


def _project(embT, wpad, vb):
    D, V = embT.shape
    return pl.pallas_call(
        _proj_body,
        grid=(pl.cdiv(_VP, vb),),
        in_specs=[
            pl.BlockSpec((D, vb), lambda i: (0, i)),
            pl.BlockSpec((D, _CP), lambda i: (0, 0)),
        ],
        out_specs=pl.BlockSpec((_CP, vb), lambda i: (0, i)),
        out_shape=jax.ShapeDtypeStruct((_CP, _VP), jnp.float32),
    )(embT, wpad)


@functools.lru_cache(maxsize=None)
def _make_il_kernel():
    nq = (_VS + _L - 1) // _L

    @functools.partial(
        pl.kernel,
        out_type=jax.ShapeDtypeStruct((_VP * _CP,), jnp.float32),
        mesh=_mesh,
        scratch_types=[
            pltpu.VMEM((_CP * _VS + _L,), jnp.float32),
            pltpu.VMEM((_VS * _CP,), jnp.float32),
        ],
        compiler_params=_sc_params,
    )
    def il_kernel(projT_hbm, out_hbm, colseg_v, ivseg_v):
        wid = lax.axis_index("s") * _NC + lax.axis_index("c")
        v0 = wid * _VS
        lane = lax.iota(jnp.int32, _L)
        for c in range(_CP):
            pltpu.sync_copy(projT_hbm.at[pl.ds(c * _VP + v0, _VS)],
                            colseg_v.at[pl.ds(c * _VS, _VS)])

        def il_body(q, _):
            r = q * _L + lane
            m = r < _VS
            for c in range(_CP):
                seg = colseg_v[pl.ds(c * _VS + q * _L, _L)]
                idx = jnp.where(m, r * _CP + c, 0)
                plsc.store_scatter(ivseg_v, [idx], seg, mask=m)
            return 0

        lax.fori_loop(0, nq, il_body, 0)
        pltpu.sync_copy(ivseg_v,
                        out_hbm.at[pl.ds(v0 * _CP, _VS * _CP)])

    return il_kernel


@functools.lru_cache(maxsize=None)
def _make_sc_kernel(T, B, V, C):
    bpw = B // _NW

    @functools.partial(
        pl.kernel,
        out_type=jax.ShapeDtypeStruct((C * B,), jnp.float32),
        mesh=_mesh,
        scratch_types=[
            pltpu.VMEM((_CH,), jnp.int32),
            pltpu.VMEM((_CH, _CP), jnp.float32),
            pltpu.VMEM((bpw + 16,), jnp.int32),
            pltpu.VMEM((C * bpw,), jnp.float32),
            pltpu.VMEM((16,), jnp.float32),
            pltpu.SemaphoreType.DMA,
        ],
        compiler_params=_sc_params,
    )
    def sc_kernel(text_hbm, offs_hbm, proj_hbm, fcb_hbm, out_hbm,
                  txt_v, rows_v, off_v, line_v, fcb_v, sem):
        wid = lax.axis_index("s") * _NC + lax.axis_index("c")
        b0 = wid * bpw
        pltpu.sync_copy(offs_hbm.at[pl.ds(b0, bpw + 16)], off_v)
        pltpu.sync_copy(fcb_hbm, fcb_v)
        lane = lax.iota(jnp.int32, _L)
        bias_vec = fcb_v[pl.ds(0, _L)]
        tq = lax.shift_right_logical(lane, 2)
        cq = lax.bitwise_and(lane, 3)
        perm8 = lax.bitwise_and(lane + 8, 15)
        perm4 = lax.bitwise_and(lane + 4, 15)
        store_idx_base = lane * bpw
        store_mask = lane < C

        def bag_body(b, cur):
            offpair = off_v[pl.ds(b, _L)]
            lo = offpair[0]
            hi = offpair[1]
            i0 = lax.div(lo, 16)
            i1 = lax.div(hi + 15, 16)

            def blk_body(i, carry):
                acc, cur = carry
                ck = lax.div(i * 16, _CH)

                @pl.when(ck != cur)
                def _():
                    pltpu.sync_copy(
                        text_hbm.at[pl.ds(ck * _CH, _CH)], txt_v)
                    pltpu.async_copy(
                        proj_hbm.at[txt_v], rows_v, sem).wait()

                local = i * 16 - ck * _CH
                for j in range(4):
                    tok = i * 16 + 4 * j + tq
                    m = (tok >= lo) & (tok < hi)
                    ridx = jnp.where(m, local + 4 * j + tq, 0)
                    vals = plsc.load_gather(rows_v, [ridx, cq], mask=m)
                    acc = acc + jnp.where(m, vals, jnp.float32(0.0))
                return (acc, ck)

            acc, cur = lax.fori_loop(
                i0, i1, blk_body,
                (jnp.zeros((_L,), jnp.float32), cur))
            acc = acc + acc.at[perm8].get(mode="promise_in_bounds")
            acc = acc + acc.at[perm4].get(mode="promise_in_bounds")
            plsc.store_scatter(
                line_v, [store_idx_base + b], acc, mask=store_mask)
            return cur

        lax.fori_loop(0, bpw, bag_body, jnp.int32(-1))

        for cl in range(C):
            bias_c = bias_vec[cl]

            def mean_body(j, _, _bias=bias_c, _cl=cl):
                sums = line_v[pl.ds(_cl * bpw + j * _L, _L)]
                o_lo = off_v[pl.ds(j * _L, _L)]
                o_hi = off_v[pl.ds(j * _L + 1, _L)]
                cnt = (o_hi - o_lo).astype(jnp.float32)
                line_v[pl.ds(_cl * bpw + j * _L, _L)] = (
                    sums / jnp.maximum(cnt, jnp.float32(1.0)) + _bias)
                return 0

            lax.fori_loop(0, bpw // _L, mean_body, 0)
            pltpu.sync_copy(line_v.at[pl.ds(cl * bpw, bpw)],
                            out_hbm.at[pl.ds(cl * B + b0, bpw)])

    return sc_kernel


def kernel(text, offsets, emb_table, fc_w, fc_b):
    T = text.shape[0]
    B = offsets.shape[0]
    V, D = emb_table.shape
    C = fc_w.shape[0]

    wpad = jnp.zeros((D, _CP), jnp.float32).at[:, :C].set(fc_w.T)
    projT = _project(emb_table.T, wpad, 12800).reshape(-1)
    proj4 = _make_il_kernel()(projT).reshape(_VP, _CP)

    offs_ext = jnp.concatenate(
        [offsets.astype(jnp.int32), jnp.full((16,), T, jnp.int32)])
    fcb_pad = jnp.zeros((16,), jnp.float32).at[:C].set(fc_b)

    out_flat = _make_sc_kernel(T, B, V, C)(
        text.astype(jnp.int32), offs_ext, proj4, fcb_pad)
    return out_flat.reshape(C, B).T


# speedup vs baseline: 1.6127x; 1.6127x over previous
import functools

import jax
import jax.numpy as jnp
from jax import lax
from jax.experimental import pallas as pl
from jax.experimental.pallas import tpu as pltpu
from jax.experimental.pallas import tpu_sc as plsc

_NC = 2
_NS = 16
_L = 16
_NW = _NC * _NS

_CP = 4
_CH = 8192
_VP = 100096
_VS = _VP // _NW

_sc_params = pltpu.CompilerParams(
    needs_layout_passes=False, use_tc_tiling_on_sc=False)
_mesh = plsc.VectorSubcoreMesh(
    core_axis_name="c", subcore_axis_name="s",
    num_cores=_NC, num_subcores=_NS)


def _proj_body(embT_ref, w_ref, out_ref):
    out_ref[...] = lax.dot_general(
        w_ref[...], embT_ref[...], (((0,), (0,)), ((), ())),
        preferred_element_type=jnp.float32)


def _project(embT, wpad, vb):
    D, V = embT.shape
    return pl.pallas_call(
        _proj_body,
        grid=(pl.cdiv(_VP, vb),),
        in_specs=[
            pl.BlockSpec((D, vb), lambda i: (0, i)),
            pl.BlockSpec((D, _CP), lambda i: (0, 0)),
        ],
        out_specs=pl.BlockSpec((_CP, vb), lambda i: (0, i)),
        out_shape=jax.ShapeDtypeStruct((_CP, _VP), jnp.float32),
    )(embT, wpad)


@functools.lru_cache(maxsize=None)
def _make_il_kernel():
    nq = (_VS + _L - 1) // _L

    @functools.partial(
        pl.kernel,
        out_type=jax.ShapeDtypeStruct((_VP, _CP), jnp.float32),
        mesh=_mesh,
        scratch_types=[
            pltpu.VMEM((_CP * _VS + _L,), jnp.float32),
            pltpu.VMEM((_VS, _CP), jnp.float32),
        ],
        compiler_params=_sc_params,
    )
    def il_kernel(projT_hbm, out_hbm, colseg_v, ivseg_v):
        wid = lax.axis_index("s") * _NC + lax.axis_index("c")
        v0 = wid * _VS
        lane = lax.iota(jnp.int32, _L)
        for c in range(_CP):
            pltpu.sync_copy(projT_hbm.at[pl.ds(c * _VP + v0, _VS)],
                            colseg_v.at[pl.ds(c * _VS, _VS)])

        def il_body(q, _):
            r = q * _L + lane
            m = r < _VS
            ridx = jnp.where(m, r, 0)
            cfull = jnp.full((_L,), 0, jnp.int32)
            for c in range(_CP):
                seg = colseg_v[pl.ds(c * _VS + q * _L, _L)]
                plsc.store_scatter(
                    ivseg_v, [ridx, cfull + c], seg, mask=m)
            return 0

        lax.fori_loop(0, nq, il_body, 0)
        pltpu.sync_copy(ivseg_v, out_hbm.at[pl.ds(v0, _VS), :])

    return il_kernel


@functools.lru_cache(maxsize=None)
def _make_sc_kernel(T, B, V, C):
    bpw = B // _NW

    @functools.partial(
        pl.kernel,
        out_type=jax.ShapeDtypeStruct((C * B,), jnp.float32),
        mesh=_mesh,
        scratch_types=[
            pltpu.VMEM((_CH,), jnp.int32),
            pltpu.VMEM((_CH, _CP), jnp.float32),
            pltpu.VMEM((bpw + 16,), jnp.int32),
            pltpu.VMEM((C * bpw,), jnp.float32),
            pltpu.VMEM((16,), jnp.float32),
            pltpu.SemaphoreType.DMA,
        ],
        compiler_params=_sc_params,
    )
    def sc_kernel(text_hbm, offs_hbm, proj_hbm, fcb_hbm, out_hbm,
                  txt_v, rows_v, off_v, line_v, fcb_v, sem):
        wid = lax.axis_index("s") * _NC + lax.axis_index("c")
        b0 = wid * bpw
        pltpu.sync_copy(offs_hbm.at[pl.ds(b0, bpw + 16)], off_v)
        pltpu.sync_copy(fcb_hbm, fcb_v)
        lane = lax.iota(jnp.int32, _L)
        bias_vec = fcb_v[pl.ds(0, _L)]
        tq = lax.shift_right_logical(lane, 2)
        cq = lax.bitwise_and(lane, 3)
        perm8 = lax.bitwise_and(lane + 8, 15)
        perm4 = lax.bitwise_and(lane + 4, 15)
        store_idx_base = lane * bpw
        store_mask = lane < C

        def bag_body(b, cur):
            offpair = off_v[pl.ds(b, _L)]
            lo = offpair[0]
            hi = offpair[1]
            i0 = lax.div(lo, 16)
            i1 = lax.div(hi + 15, 16)

            def blk_body(i, carry):
                acc, cur = carry
                ck = lax.div(i * 16, _CH)

                @pl.when(ck != cur)
                def _():
                    pltpu.sync_copy(
                        text_hbm.at[pl.ds(ck * _CH, _CH)], txt_v)
                    pltpu.async_copy(
                        proj_hbm.at[txt_v], rows_v, sem).wait()

                local = i * 16 - ck * _CH
                for j in range(4):
                    tok = i * 16 + 4 * j + tq
                    m = (tok >= lo) & (tok < hi)
                    ridx = jnp.where(m, local + 4 * j + tq, 0)
                    vals = plsc.load_gather(rows_v, [ridx, cq], mask=m)
                    acc = acc + jnp.where(m, vals, jnp.float32(0.0))
                return (acc, ck)

            acc, cur = lax.fori_loop(
                i0, i1, blk_body,
                (jnp.zeros((_L,), jnp.float32), cur))
            acc = acc + acc.at[perm8].get(mode="promise_in_bounds")
            acc = acc + acc.at[perm4].get(mode="promise_in_bounds")
            plsc.store_scatter(
                line_v, [store_idx_base + b], acc, mask=store_mask)
            return cur

        lax.fori_loop(0, bpw, bag_body, jnp.int32(-1))

        for cl in range(C):
            bias_c = bias_vec[cl]

            def mean_body(j, _, _bias=bias_c, _cl=cl):
                sums = line_v[pl.ds(_cl * bpw + j * _L, _L)]
                o_lo = off_v[pl.ds(j * _L, _L)]
                o_hi = off_v[pl.ds(j * _L + 1, _L)]
                cnt = (o_hi - o_lo).astype(jnp.float32)
                line_v[pl.ds(_cl * bpw + j * _L, _L)] = (
                    sums / jnp.maximum(cnt, jnp.float32(1.0)) + _bias)
                return 0

            lax.fori_loop(0, bpw // _L, mean_body, 0)
            pltpu.sync_copy(line_v.at[pl.ds(cl * bpw, bpw)],
                            out_hbm.at[pl.ds(cl * B + b0, bpw)])

    return sc_kernel


def kernel(text, offsets, emb_table, fc_w, fc_b):
    T = text.shape[0]
    B = offsets.shape[0]
    V, D = emb_table.shape
    C = fc_w.shape[0]

    wpad = jnp.zeros((D, _CP), jnp.float32).at[:, :C].set(fc_w.T)
    projT = _project(emb_table.T, wpad, 12800).reshape(-1)
    proj4 = _make_il_kernel()(projT)

    offs_ext = jnp.concatenate(
        [offsets.astype(jnp.int32), jnp.full((16,), T, jnp.int32)])
    fcb_pad = jnp.zeros((16,), jnp.float32).at[:C].set(fc_b)

    out_flat = _make_sc_kernel(T, B, V, C)(
        text.astype(jnp.int32), offs_ext, proj4, fcb_pad)
    return out_flat.reshape(C, B).T
